# baseline (speedup 1.0000x reference)
import jax
import jax.numpy as jnp
import numpy as np
from jax import lax
from jax.experimental import pallas as pl
from jax.experimental.pallas import tpu as pltpu
from jax.experimental.pallas import tpu_sc as plsc

B = 4096
M = 20
J = 21
L = 64
K_CONV = 25
H_OUT = L - K_CONV + 1
O_CONV = 5
N_FEAT = O_CONV * H_OUT
NUM_USERS = 117362
ROWS = NUM_USERS + 1
JL = J * L

NC = 2
NS = 16
NW = NC * NS
CHUNK = B // NW

BB = 512
GRID = B // BB

def _wid():
    return lax.axis_index("s") * NC + lax.axis_index("c")


def _sc_gather_body(uid_hbm, comb_hbm, out_hbm, idx_v, rows_v, sem):
    base = _wid() * CHUNK
    pltpu.sync_copy(uid_hbm.at[pl.ds(base, CHUNK)], idx_v)
    pltpu.async_copy(comb_hbm.at[idx_v], rows_v, sem).wait()
    pltpu.sync_copy(rows_v, out_hbm.at[pl.ds(base, CHUNK)])


import functools


@functools.cache
def _sc_kernels():
    mesh = plsc.VectorSubcoreMesh(
        core_axis_name="c", subcore_axis_name="s",
        num_cores=NC, num_subcores=NS)
    params = pltpu.CompilerParams(use_tc_tiling_on_sc=False)
    gather = pl.kernel(
        _sc_gather_body,
        out_type=jax.ShapeDtypeStruct((B, 8), jnp.float32),
        mesh=mesh,
        compiler_params=params,
        scratch_types=[
            pltpu.VMEM((CHUNK,), jnp.int32),
            pltpu.VMEM((CHUNK, 8), jnp.float32),
            pltpu.SemaphoreType.DMA,
        ],
    )
    return gather


def _patch_body(uid_ref, lo_ref, feat_ref, tbl_ref, out_ref, psem):
    def body(k, carry):
        u = uid_ref[k]
        src = lo_ref[k]
        pltpu.make_async_copy(feat_ref.at[pl.ds(src, 1), :],
                              out_ref.at[pl.ds(u, 1), :], psem).start()
        return carry

    lax.fori_loop(0, B, body, 0, unroll=16)

    pltpu.make_async_copy(feat_ref, out_ref.at[pl.ds(0, B), :], psem).wait()


_tc_scatter = pl.pallas_call(
    _patch_body,
    in_specs=[
        pl.BlockSpec(memory_space=pltpu.SMEM),
        pl.BlockSpec(memory_space=pltpu.SMEM),
        pl.BlockSpec(memory_space=pltpu.VMEM),
        pl.BlockSpec(memory_space=pl.ANY),
    ],
    out_specs=pl.BlockSpec(memory_space=pl.ANY),
    out_shape=jax.ShapeDtypeStruct((ROWS, N_FEAT), jnp.float32),
    input_output_aliases={3: 0},
    scratch_shapes=[pltpu.SemaphoreType.DMA],
)


def _tc_body(uid_ref, macro_ref, micro_ref, gath_ref, w2_ref, e_ref, cb_ref,
             ub_ref, mu_ref, feat_ref, out_ref, lo_ref, acc_ref):
    g = pl.program_id(0)

    s = jnp.sum(gath_ref[:, 0:4], axis=1)
    msum = jnp.sum(micro_ref[...], axis=1)
    c = s[:, None] * msum * (1.0 / M)

    cexp = jnp.dot(c, e_ref[...], preferred_element_type=jnp.float32)
    a = macro_ref[...] * cexp
    pre = jnp.dot(a, w2_ref[...], preferred_element_type=jnp.float32)
    feat = jnp.maximum(pre + cb_ref[0, :][None, :], 0.0)
    feat_ref[...] = feat

    uid_all = uid_ref[0, :]
    uid_blk = uid_ref[0, pl.ds(g * BB, BB)]
    eq = uid_blk[:, None] == uid_all[None, :]
    jidx = lax.broadcasted_iota(jnp.int32, (BB, B), 1)
    lo_ref[0, pl.ds(g * BB, BB)] = jnp.max(jnp.where(eq, jidx, -1), axis=1)

    psum = jnp.sum(feat)
    total = jnp.where(g == 0, psum, acc_ref[0] + psum)
    acc_ref[0] = total

    @pl.when(g == GRID - 1)
    def _():
        out_ref[0, :] = total + ub_ref[0, :] + mu_ref[0, 0]


_tc_compute = pl.pallas_call(
    _tc_body,
    grid=(GRID,),
    in_specs=[
        pl.BlockSpec((1, B), lambda g: (0, 0)),
        pl.BlockSpec((BB, JL), lambda g: (g, 0)),
        pl.BlockSpec((BB, M, J), lambda g: (g, 0, 0)),
        pl.BlockSpec((BB, 8), lambda g: (g, 0)),
        pl.BlockSpec((JL, N_FEAT), lambda g: (0, 0)),
        pl.BlockSpec((J, JL), lambda g: (0, 0)),
        pl.BlockSpec((1, N_FEAT), lambda g: (0, 0)),
        pl.BlockSpec((1, B), lambda g: (0, 0)),
        pl.BlockSpec((1, 1), lambda g: (0, 0)),
    ],
    out_specs=[
        pl.BlockSpec((BB, N_FEAT), lambda g: (g, 0)),
        pl.BlockSpec((1, B), lambda g: (0, 0)),
        pl.BlockSpec((1, B), lambda g: (0, 0)),
    ],
    out_shape=[
        jax.ShapeDtypeStruct((B, N_FEAT), jnp.float32),
        jax.ShapeDtypeStruct((1, B), jnp.float32),
        jax.ShapeDtypeStruct((1, B), jnp.int32),
    ],
    scratch_shapes=[pltpu.SMEM((1,), jnp.float32)],
)


def _build_w2(conv_w):
    k = np.arange(L)
    h = np.arange(H_OUT)
    d = k[:, None] - h[None, :]
    valid = jnp.asarray((d >= 0) & (d < K_CONV))
    dc = np.clip(d, 0, K_CONV - 1)
    w = conv_w[:, :, dc]
    w = jnp.where(valid[None, None], w, 0.0)
    return w.transpose(1, 2, 0, 3).reshape(JL, N_FEAT)


def kernel(macro, micro, prop_pref_table, conv_w, conv_b, user_bias_table,
           user_embedding_table, mu_bias, user_id):
    sc_gather = _sc_kernels()
    uid = user_id.astype(jnp.int32)
    comb = jnp.concatenate(
        [prop_pref_table, user_bias_table,
         jnp.zeros((ROWS, 3), jnp.float32)], axis=1)
    gath = sc_gather(uid, comb)

    macro2d = macro.reshape(B, JL)
    w2 = _build_w2(conv_w)
    cb = jnp.repeat(conv_b, H_OUT)[None, :]
    e = jnp.asarray(
        np.equal.outer(np.arange(J), np.arange(JL) // L).astype(np.float32))
    ub2d = gath[:, 4][None, :]
    mu2d = mu_bias[None, :]

    uid2d = uid[None, :]
    feat, outv, lo = _tc_compute(uid2d, macro2d, micro, gath, w2, e, cb,
                                 ub2d, mu2d)

    updated = _tc_scatter(uid, lo.reshape(B), feat, user_embedding_table)
    return outv.reshape(B), updated
